# Initial kernel scaffold; baseline (speedup 1.0000x reference)
import jax
import jax.numpy as jnp
from jax.experimental import pallas as pl


def kernel(x1, x2, gamma, beta):
    raise NotImplementedError("write your pallas kernel here")


# speedup vs baseline: 2.3162x; 2.3162x over previous
import functools

import jax
import jax.numpy as jnp
from jax import lax
from jax.experimental import pallas as pl
from jax.experimental.pallas import tpu as pltpu
from jax.experimental.pallas import tpu_sc as plsc

_B, _C, _H, _W = 16, 384, 56, 56
_HW = _H * _W
_C2 = 2 * _C
_N = _B * _HW
_EPS = 1e-5
_CB = 64
_NB = 4
_NSC = 32
_RPT = (_B * _C) // _NSC
_LANES = 16


def _col_to_row(col):
    n = col.shape[0]
    ci = lax.broadcasted_iota(jnp.int32, (n, n), 0)
    cj = lax.broadcasted_iota(jnp.int32, (n, n), 1)
    return jnp.sum(jnp.where(ci == cj, col, 0.0), axis=0, keepdims=True)


def _row_to_col(row):
    n = row.shape[1]
    ci = lax.broadcasted_iota(jnp.int32, (n, n), 0)
    cj = lax.broadcasted_iota(jnp.int32, (n, n), 1)
    return jnp.sum(jnp.where(ci == cj, row, 0.0), axis=1, keepdims=True)


def _stats_body(x1_ref, x2_ref, s1_ref, q1_ref, s2_ref, q2_ref):
    x1 = x1_ref[0]
    x2 = x2_ref[0]
    s1_ref[...] = _col_to_row(jnp.sum(x1, axis=1, keepdims=True)).reshape(
        1, 1, 1, _CB)
    q1_ref[...] = _col_to_row(jnp.sum(x1 * x1, axis=1, keepdims=True)).reshape(
        1, 1, 1, _CB)
    s2_ref[...] = _col_to_row(jnp.sum(x2, axis=1, keepdims=True)).reshape(
        1, 1, 1, _CB)
    q2_ref[...] = _col_to_row(jnp.sum(x2 * x2, axis=1, keepdims=True)).reshape(
        1, 1, 1, _CB)


def _stats(x1r, x2r):
    out = jax.ShapeDtypeStruct((_B, _C // _CB, 1, _CB), jnp.float32)
    return pl.pallas_call(
        _stats_body,
        grid=(_B, _C // _CB),
        in_specs=[
            pl.BlockSpec((1, _CB, _HW), lambda b, cb: (b, cb, 0)),
            pl.BlockSpec((1, _CB, _HW), lambda b, cb: (b, cb, 0)),
        ],
        out_specs=[pl.BlockSpec((1, 1, 1, _CB),
                                lambda b, cb: (b, cb, 0, 0))] * 4,
        out_shape=[out] * 4,
    )(x1r, x2r)


def _select_body(s1_ref, q1_ref, s2_ref, q2_ref, g_ref, b_ref,
                 idx_ref, sc_ref, sh_ref):
    ci = lax.broadcasted_iota(jnp.int32, (_C2, _C2), 0)
    cj = lax.broadcasted_iota(jnp.int32, (_C2, _C2), 1)
    jrow = lax.broadcasted_iota(jnp.int32, (_C2, _C), 1)
    cval = lax.broadcasted_iota(jnp.int32, (_C2, _C), 0)

    def per_b(b, carry):
        ssum, qsum = carry
        vrow = jnp.concatenate(
            [s1_ref[pl.ds(b, 1), :], s2_ref[pl.ds(b, 1), :]], axis=1)
        qrow = jnp.concatenate(
            [q1_ref[pl.ds(b, 1), :], q2_ref[pl.ds(b, 1), :]], axis=1)
        vcol = _row_to_col(vrow)
        qcol = _row_to_col(qrow)
        beats = (vrow > vcol) | ((vrow == vcol) & (cj < ci))
        rank = jnp.sum(jnp.where(beats, 1, 0), axis=1, keepdims=True)
        iseq = rank == jrow
        onehot = jnp.where(iseq, 1.0, 0.0)
        ssum = ssum + jnp.sum(onehot * vcol, axis=0, keepdims=True)
        qsum = qsum + jnp.sum(onehot * qcol, axis=0, keepdims=True)
        idxrow = jnp.sum(jnp.where(iseq, cval, 0), axis=0, keepdims=True)
        idx_ref[pl.ds(b, 1), :] = idxrow
        return ssum, qsum

    zero = jnp.zeros((1, _C), jnp.float32)
    ssum, qsum = lax.fori_loop(0, _B, per_b, (zero, zero))
    mean = ssum * (1.0 / _N)
    var = qsum * (1.0 / _N) - mean * mean
    scale = g_ref[...] * lax.rsqrt(var + _EPS)
    sc_ref[...] = scale
    sh_ref[...] = b_ref[...] - mean * scale


def _select(s1, q1, s2, q2, gamma, beta):
    mat = pl.BlockSpec((_B, _C), lambda: (0, 0))
    vec = pl.BlockSpec((1, _C), lambda: (0, 0))
    return pl.pallas_call(
        _select_body,
        in_specs=[mat, mat, mat, mat, vec, vec],
        out_specs=[
            pl.BlockSpec((_B, _C), lambda: (0, 0)),
            vec,
            vec,
        ],
        out_shape=[
            jax.ShapeDtypeStruct((_B, _C), jnp.int32),
            jax.ShapeDtypeStruct((1, _C), jnp.float32),
            jax.ShapeDtypeStruct((1, _C), jnp.float32),
        ],
    )(s1, q1, s2, q2, gamma, beta)


def _gather_body(x1_ref, x2_ref, idx_ref, sc_ref, sh_ref, out_ref,
                 idx_v, sc_v, sh_v, gbuf, sbuf, gsem, ssem):
    cid = lax.axis_index("c")
    sid = lax.axis_index("s")
    b = sid
    j0 = cid * _RPT

    pltpu.sync_copy(idx_ref.at[b, pl.ds(j0, _RPT)], idx_v)
    pltpu.sync_copy(sc_ref.at[pl.ds(j0, _RPT)], sc_v)
    pltpu.sync_copy(sh_ref.at[pl.ds(j0, _RPT)], sh_v)

    def issue_gather(i, g):
        c = idx_v[i][0]
        @pl.when(c < _C)
        def _():
            pltpu.async_copy(x1_ref.at[b, c], gbuf.at[g], gsem.at[g])
        @pl.when(c >= _C)
        def _():
            pltpu.async_copy(x2_ref.at[b, c - _C], gbuf.at[g], gsem.at[g])

    for g in range(_NB):
        issue_gather(g, g)

    def outer(t, carry):
        for g in range(_NB):
            i = t * _NB + g
            pltpu.make_async_copy(x1_ref.at[0, 0], gbuf.at[g],
                                  gsem.at[g]).wait()
            @pl.when(t > 0)
            def _():
                pltpu.make_async_copy(sbuf.at[g], out_ref.at[0, 0],
                                      ssem.at[g]).wait()
            scv = sc_v[i]
            shv = sh_v[i]
            def ew(k, c, g=g, scv=scv, shv=shv):
                x = gbuf[g, pl.ds(k * _LANES, _LANES)]
                y = jnp.maximum(x * scv + shv, 0.0)
                sbuf[g, pl.ds(k * _LANES, _LANES)] = y
                return c
            lax.fori_loop(0, _HW // _LANES, ew, 0, unroll=7)
            pltpu.async_copy(sbuf.at[g], out_ref.at[b, j0 + i], ssem.at[g])
            @pl.when(i + _NB < _RPT)
            def _():
                issue_gather(i + _NB, g)
        return carry
    lax.fori_loop(0, _RPT // _NB, outer, 0)

    for g in range(_NB):
        pltpu.make_async_copy(sbuf.at[g], out_ref.at[0, 0], ssem.at[g]).wait()


def _gather(x1r, x2r, idx16, sc16, sh16):
    mesh = plsc.VectorSubcoreMesh(core_axis_name="c", subcore_axis_name="s")
    fn = pl.kernel(
        _gather_body,
        out_type=jax.ShapeDtypeStruct((_B, _C, _HW), jnp.float32),
        mesh=mesh,
        scratch_types=[
            pltpu.VMEM((_RPT, _LANES), jnp.int32),
            pltpu.VMEM((_RPT, _LANES), jnp.float32),
            pltpu.VMEM((_RPT, _LANES), jnp.float32),
            pltpu.VMEM((_NB, _HW), jnp.float32),
            pltpu.VMEM((_NB, _HW), jnp.float32),
            pltpu.SemaphoreType.DMA((_NB,)),
            pltpu.SemaphoreType.DMA((_NB,)),
        ],
    )
    return fn(x1r, x2r, idx16, sc16, sh16)


def kernel(x1, x2, gamma, beta):
    x1r = x1.reshape(_B, _C, _HW)
    x2r = x2.reshape(_B, _C, _HW)
    s1, q1, s2, q2 = (a.reshape(_B, _C) for a in _stats(x1r, x2r))
    idx, scale, shift = _select(s1, q1, s2, q2,
                                gamma.reshape(1, _C), beta.reshape(1, _C))
    idx16 = jnp.broadcast_to(idx[:, :, None], (_B, _C, _LANES))
    sc16 = jnp.broadcast_to(scale.reshape(_C, 1), (_C, _LANES))
    sh16 = jnp.broadcast_to(shift.reshape(_C, 1), (_C, _LANES))
    out = _gather(x1r, x2r, idx16, sc16, sh16)
    return out.reshape(_B, _C, _H, _W)
